# Initial kernel scaffold; baseline (speedup 1.0000x reference)
import jax
import jax.numpy as jnp
from jax.experimental import pallas as pl


def kernel(x, edge_index, alpha, bias, W, b_lin, gamma, beta):
    raise NotImplementedError("write your pallas kernel here")


# speedup vs baseline: 7.1117x; 7.1117x over previous
import functools

import jax
import jax.numpy as jnp
from jax import lax
from jax.experimental import pallas as pl
from jax.experimental.pallas import tpu as pltpu
from jax.experimental.pallas import tpu_sc as plsc

N = 50000
E = 1600000
B = 32
NUM_LABELS = 20

NC = 2
NS = 16
NW = NC * NS
L = 16

K = 128
CH = -(-E // (NW * K))
E_PAD = NW * K * CH
N_PAD = 50176
ZROWS = N_PAD // NS

_f32 = jnp.float32
_i32 = jnp.int32


def _sc_body(xT, srcs, dsts, alphas, biases, z32, z8, ones8,
             out_acc, out_cnt,
             acc_sh, cnt_sh, src_v, dst_v, al_v, bi_v, rows_v, ones_v, sem):
    c = lax.axis_index("c")
    s = lax.axis_index("s")
    wid = c * NS + s

    zbase = s * ZROWS
    pltpu.sync_copy(z32, acc_sh.at[pl.ds(zbase, ZROWS)])
    pltpu.sync_copy(z8, cnt_sh.at[pl.ds(zbase, ZROWS)])
    pltpu.sync_copy(ones8, ones_v)
    plsc.subcore_barrier()

    def chunk(ci, carry):
        eb = (wid * CH + ci) * K
        pltpu.sync_copy(srcs.at[pl.ds(eb, K)], src_v)
        pltpu.sync_copy(dsts.at[pl.ds(eb, K)], dst_v)
        pltpu.sync_copy(alphas.at[pl.ds(eb, K)], al_v)
        pltpu.sync_copy(biases.at[pl.ds(eb, K)], bi_v)
        pltpu.async_copy(xT.at[src_v], rows_v, sem).wait()

        def group(g, carry2):
            a16 = al_v[pl.ds(g * L, L)]
            b16 = bi_v[pl.ds(g * L, L)]
            for j in range(L):
                e = g * L + j
                sel = jnp.full((L,), j, dtype=_i32)
                av = a16.at[sel].get(mode="promise_in_bounds")
                bv = b16.at[sel].get(mode="promise_in_bounds")
                r0 = rows_v[e, pl.ds(0, L)]
                r1 = rows_v[e, pl.ds(L, L)]
                rows_v[e, pl.ds(0, L)] = r0 * av + bv
                rows_v[e, pl.ds(L, L)] = r1 * av + bv
            return carry2

        lax.fori_loop(0, K // L, group, 0)

        pltpu.sync_copy(rows_v, acc_sh.at[dst_v], add=True)
        pltpu.sync_copy(ones_v, cnt_sh.at[dst_v], add=True)
        return carry

    lax.fori_loop(0, CH, chunk, 0)
    plsc.subcore_barrier()

    pltpu.sync_copy(acc_sh.at[pl.ds(zbase, ZROWS)],
                    out_acc.at[c, pl.ds(zbase, ZROWS)])
    pltpu.sync_copy(cnt_sh.at[pl.ds(zbase, ZROWS)],
                    out_cnt.at[c, pl.ds(zbase, ZROWS)])


_sc_call = functools.partial(
    pl.kernel,
    out_type=[jax.ShapeDtypeStruct((NC, N_PAD, B), _f32),
              jax.ShapeDtypeStruct((NC, N_PAD, 8), _f32)],
    mesh=plsc.VectorSubcoreMesh(core_axis_name="c", subcore_axis_name="s",
                                num_cores=NC, num_subcores=NS),
    scratch_types=[
        pltpu.VMEM_SHARED((N_PAD, B), _f32),
        pltpu.VMEM_SHARED((N_PAD, 8), _f32),
        pltpu.VMEM((K,), _i32),
        pltpu.VMEM((K,), _i32),
        pltpu.VMEM((K,), _f32),
        pltpu.VMEM((K,), _f32),
        pltpu.VMEM((K, B), _f32),
        pltpu.VMEM((K, 8), _f32),
        pltpu.SemaphoreType.DMA,
    ],
    compiler_params=pltpu.CompilerParams(use_tc_tiling_on_sc=False),
)(_sc_body)


def _tr_body(x_ref, out_ref):
    r = lax.broadcasted_iota(_i32, (B, B), 0)
    cidx = lax.broadcasted_iota(_i32, (B, B), 1)
    eye = (r == cidx).astype(_f32)
    out_ref[...] = jax.lax.dot_general(
        x_ref[...], eye, (((0,), (0,)), ((), ())),
        preferred_element_type=_f32)


def _transpose_x(xp):
    return pl.pallas_call(
        _tr_body,
        out_shape=jax.ShapeDtypeStruct((N_PAD, B), _f32),
    )(xp)


TBLK = 1792
NB = N_PAD // TBLK


def _fin_body(acc_ref, cnt_ref, w_ref, g_ref, b_ref, bn_ref, pred_ref, pacc):
    i = pl.program_id(0)
    sums = acc_ref[0] + acc_ref[1]
    counts = cnt_ref[0, :, 0:1] + cnt_ref[1, :, 0:1]
    mean = sums / jnp.maximum(counts, 1.0)
    th = jnp.tanh(mean)
    mu = jnp.mean(th, axis=1, keepdims=True)
    var = jnp.mean((th - mu) * (th - mu), axis=1, keepdims=True)
    bn = (th - mu) / jnp.sqrt(var + 1e-5) * g_ref[...] + b_ref[...]

    r = lax.broadcasted_iota(_i32, (B, B), 0)
    cidx = lax.broadcasted_iota(_i32, (B, B), 1)
    eye = (r == cidx).astype(_f32)
    bn_ref[...] = jax.lax.dot_general(
        eye, bn, (((1,), (1,)), ((), ())), preferred_element_type=_f32)

    @pl.when(i == 0)
    def _():
        pacc[...] = jnp.zeros((B, B), _f32)

    pacc[...] += jax.lax.dot_general(
        w_ref[...], mean, (((1,), (0,)), ((), ())),
        preferred_element_type=_f32)

    @pl.when(i == NB - 1)
    def _():
        pred_ref[...] = pacc[...]


def _final(acc, cnt, Wp, g2, b2):
    return pl.pallas_call(
        _fin_body,
        grid=(NB,),
        in_specs=[
            pl.BlockSpec((NC, TBLK, B), lambda i: (0, i, 0)),
            pl.BlockSpec((NC, TBLK, 8), lambda i: (0, i, 0)),
            pl.BlockSpec((B, TBLK), lambda i: (0, i)),
            pl.BlockSpec((TBLK, 1), lambda i: (i, 0)),
            pl.BlockSpec((TBLK, 1), lambda i: (i, 0)),
        ],
        out_specs=[
            pl.BlockSpec((B, TBLK), lambda i: (0, i)),
            pl.BlockSpec((B, B), lambda i: (0, 0)),
        ],
        out_shape=[
            jax.ShapeDtypeStruct((B, N_PAD), _f32),
            jax.ShapeDtypeStruct((B, B), _f32),
        ],
        scratch_shapes=[pltpu.VMEM((B, B), _f32)],
    )(acc, cnt, Wp, g2, b2)


def kernel(x, edge_index, alpha, bias, W, b_lin, gamma, beta):
    src = edge_index[0]
    dst = edge_index[1]
    pad = E_PAD - E
    srcs = jnp.concatenate([src, jnp.zeros((pad,), _i32)])
    dsts = jnp.concatenate([dst, jnp.full((pad,), N, _i32)])
    alphas = jnp.concatenate([alpha, jnp.zeros((pad,), _f32)])
    biases = jnp.concatenate([bias, jnp.zeros((pad,), _f32)])

    xp = jnp.pad(x, ((0, 0), (0, N_PAD - N)))
    xT = _transpose_x(xp)

    z32 = jnp.zeros((ZROWS, B), _f32)
    z8 = jnp.zeros((ZROWS, 8), _f32)
    ones8 = jnp.tile(jnp.array([1, 0, 0, 0, 0, 0, 0, 0], _f32)[None, :],
                     (K, 1))

    acc, cnt = _sc_call(xT, srcs, dsts, alphas, biases, z32, z8, ones8)

    Wp = jnp.pad(W, ((0, B - NUM_LABELS), (0, N_PAD - N)))
    g2 = jnp.pad(gamma, (0, N_PAD - N))[:, None]
    b2 = jnp.pad(beta, (0, N_PAD - N))[:, None]

    bn_full, pred32 = _final(acc, cnt, Wp, g2, b2)

    pred = pred32.T[:, :NUM_LABELS] + b_lin[None, :]
    bn = bn_full[:, :N]
    return (pred, bn)
